# baseline (speedup 1.0000x reference)
import functools

import jax
import jax.numpy as jnp
import numpy as np
from jax import lax
from jax.experimental import pallas as pl
from jax.experimental.pallas import tpu as pltpu
from jax.experimental.pallas import tpu_sc as plsc

SPARSITY = 0.95
ROW_BLOCK = 256
SC_ROWS = 1536
_NW = 32

_I16_MIN = np.int16(-(2**15))


def _count16(mask):
    c = mask.astype(jnp.int16)
    w = c.shape[1]
    while w > 128:
        c = c[:, : w // 2] + c[:, w // 2 :]
        w //= 2
    return jnp.sum(c.astype(jnp.int32), axis=1, keepdims=True)


def _topk_mask_kernel(x_ref, o_ref, *, k: int):
    xb = x_ref[...]
    s = jax.lax.bitcast_convert_type(xb, jnp.int32)

    h0 = (s >> 16).astype(jnp.int16)
    neg = h0 < 0
    hi = jnp.where(neg, h0 ^ np.int16(0x7FFF), h0)
    l0 = ((s & np.int32(0xFFFF)) - 32768).astype(jnp.int16)
    lo = jnp.where(neg, l0 ^ np.int16(-1), l0)

    pu = jnp.zeros((xb.shape[0], 1), jnp.int32)
    for b in range(15, -1, -1):
        t = pu | np.int32(1 << b)
        pv = (t - 32768).astype(jnp.int16)
        cnt = _count16(hi >= pv)
        pu = jnp.where(cnt >= k, t, pu)
    thr_hi = (pu - 32768).astype(jnp.int16)

    c_gt = _count16(hi > thr_hi)
    w = jnp.where(hi == thr_hi, lo, _I16_MIN)

    pl2 = jnp.zeros((xb.shape[0], 1), jnp.int32)
    for b in range(15, -1, -1):
        t = pl2 | np.int32(1 << b)
        pv = (t - 32768).astype(jnp.int16)
        cnt = _count16(w >= pv)
        pl2 = jnp.where(c_gt + cnt >= k, t, pl2)

    thr_v = ((pu - 32768) << 16) | pl2
    thr_s = jnp.where(thr_v < 0, thr_v ^ np.int32(0x7FFFFFFF), thr_v)
    thr_f = jax.lax.bitcast_convert_type(thr_s, jnp.float32)
    o_ref[...] = jnp.where(xb >= thr_f, xb, jnp.float32(0.0))


def _apply_thr_kernel(x_ref, t_ref, _o1_ref, o_ref):
    o_ref[...] = jnp.where(x_ref[...] >= t_ref[:, :1], x_ref[...],
                           jnp.float32(0.0))


def _sc_thr_body(x_hbm, o_hbm, v_v, x_v, t_v, *, k, rpw, d, row0):
    wid = lax.axis_index("s") * 2 + lax.axis_index("c")
    base = row0 + wid * rpw
    nv = d // 16

    def row_body(r, _):
        pltpu.sync_copy(x_hbm.at[base + r], x_v)

        def enc(i, _):
            for u in range(8):
                o = i * 128 + u * 16
                s = lax.bitcast_convert_type(x_v[pl.ds(o, 16)], jnp.int32)
                v_v[pl.ds(o, 16)] = jnp.where(
                    s < 0, s ^ np.int32(0x7FFFFFFF), s)
            return 0

        lax.fori_loop(0, nv // 8, enc, 0)

        def bit_body(b, pu):
            t = pu | lax.shift_left(np.int32(1), np.int32(31) - b)
            pv = t ^ np.int32(-(2**31))

            def cnt_chunk(j, accs):
                a0, a1, a2, a3 = accs
                for u in range(0, 16, 4):
                    base2 = j * 256 + u * 16
                    a0 = a0 + plsc.all_reduce_population_count(
                        v_v[pl.ds(base2, 16)] >= pv)
                    a1 = a1 + plsc.all_reduce_population_count(
                        v_v[pl.ds(base2 + 16, 16)] >= pv)
                    a2 = a2 + plsc.all_reduce_population_count(
                        v_v[pl.ds(base2 + 32, 16)] >= pv)
                    a3 = a3 + plsc.all_reduce_population_count(
                        v_v[pl.ds(base2 + 48, 16)] >= pv)
                return a0, a1, a2, a3

            z = jnp.zeros((16,), jnp.int32)
            a0, a1, a2, a3 = lax.fori_loop(0, nv // 16, cnt_chunk,
                                           (z, z, z, z))
            cnt = (a0 + a1) + (a2 + a3)
            return jnp.where(cnt >= k, t, pu)

        pu = lax.fori_loop(0, 32, bit_body, jnp.zeros((16,), jnp.int32))
        pv = pu ^ np.int32(-(2**31))
        thr_s = jnp.where(pv < 0, pv ^ np.int32(0x7FFFFFFF), pv)
        thr_f = lax.bitcast_convert_type(thr_s, jnp.float32)
        for u in range(8):
            t_v[pl.ds(u * 16, 16)] = thr_f
        pltpu.sync_copy(t_v, o_hbm.at[wid * rpw + r])
        return 0

    lax.fori_loop(0, rpw, row_body, 0)


def kernel(x):
    d = x.shape[-1]
    k = max(1, int(d * (1.0 - SPARSITY)))
    flat = x.reshape(-1, d)
    rows = flat.shape[0]
    tc_rows = rows - SC_ROWS
    tc_blocks = tc_rows // ROW_BLOCK
    sc_blocks = SC_ROWS // ROW_BLOCK

    thr = pl.kernel(
        functools.partial(_sc_thr_body, k=k, rpw=SC_ROWS // _NW, d=d,
                          row0=tc_rows),
        out_type=jax.ShapeDtypeStruct((SC_ROWS, 128), jnp.float32),
        mesh=plsc.VectorSubcoreMesh(core_axis_name="c", subcore_axis_name="s"),
        compiler_params=pltpu.CompilerParams(needs_layout_passes=False),
        scratch_types=[
            pltpu.VMEM((d,), jnp.int32),
            pltpu.VMEM((d,), jnp.float32),
            pltpu.VMEM((128,), jnp.float32),
        ],
    )(flat)

    o1 = pl.pallas_call(
        functools.partial(_topk_mask_kernel, k=k),
        grid=(tc_blocks,),
        in_specs=[pl.BlockSpec((ROW_BLOCK, d), lambda i: (i, 0))],
        out_specs=pl.BlockSpec((ROW_BLOCK, d), lambda i: (i, 0)),
        out_shape=jax.ShapeDtypeStruct((rows, d), jnp.float32),
        compiler_params=pltpu.CompilerParams(
            dimension_semantics=("arbitrary",),
        ),
    )(flat)

    out = pl.pallas_call(
        _apply_thr_kernel,
        grid=(sc_blocks,),
        in_specs=[
            pl.BlockSpec((ROW_BLOCK, d), lambda i, n=tc_blocks: (i + n, 0)),
            pl.BlockSpec((ROW_BLOCK, 128), lambda i: (i, 0)),
            pl.BlockSpec((ROW_BLOCK, d), lambda i, n=tc_blocks: (i + n, 0)),
        ],
        out_specs=pl.BlockSpec((ROW_BLOCK, d),
                               lambda i, n=tc_blocks: (i + n, 0)),
        out_shape=jax.ShapeDtypeStruct((rows, d), jnp.float32),
        input_output_aliases={2: 0},
    )(flat, thr, o1)
    return out.reshape(x.shape)


# speedup vs baseline: 1.0782x; 1.0782x over previous
import functools

import jax
import jax.numpy as jnp
import numpy as np
from jax import lax
from jax.experimental import pallas as pl
from jax.experimental.pallas import tpu as pltpu
from jax.experimental.pallas import tpu_sc as plsc

SPARSITY = 0.95
ROW_BLOCK = 256
SC_ROWS = 1280
_NW = 32

_I16_MIN = np.int16(-(2**15))


def _count16(mask):
    c = mask.astype(jnp.int16)
    w = c.shape[1]
    while w > 128:
        c = c[:, : w // 2] + c[:, w // 2 :]
        w //= 2
    return jnp.sum(c.astype(jnp.int32), axis=1, keepdims=True)


def _topk_mask_kernel(x_ref, o_ref, *, k: int):
    xb = x_ref[...]
    s = jax.lax.bitcast_convert_type(xb, jnp.int32)

    h0 = (s >> 16).astype(jnp.int16)
    neg = h0 < 0
    hi = jnp.where(neg, h0 ^ np.int16(0x7FFF), h0)
    l0 = ((s & np.int32(0xFFFF)) - 32768).astype(jnp.int16)
    lo = jnp.where(neg, l0 ^ np.int16(-1), l0)

    pu = jnp.zeros((xb.shape[0], 1), jnp.int32)
    for b in range(15, -1, -1):
        t = pu | np.int32(1 << b)
        pv = (t - 32768).astype(jnp.int16)
        cnt = _count16(hi >= pv)
        pu = jnp.where(cnt >= k, t, pu)
    thr_hi = (pu - 32768).astype(jnp.int16)

    c_gt = _count16(hi > thr_hi)
    w = jnp.where(hi == thr_hi, lo, _I16_MIN)

    pl2 = jnp.zeros((xb.shape[0], 1), jnp.int32)
    for b in range(15, -1, -1):
        t = pl2 | np.int32(1 << b)
        pv = (t - 32768).astype(jnp.int16)
        cnt = _count16(w >= pv)
        pl2 = jnp.where(c_gt + cnt >= k, t, pl2)

    thr_v = ((pu - 32768) << 16) | pl2
    thr_s = jnp.where(thr_v < 0, thr_v ^ np.int32(0x7FFFFFFF), thr_v)
    thr_f = jax.lax.bitcast_convert_type(thr_s, jnp.float32)
    o_ref[...] = jnp.where(xb >= thr_f, xb, jnp.float32(0.0))


def _apply_thr_kernel(x_ref, t_ref, _o1_ref, o_ref):
    o_ref[...] = jnp.where(x_ref[...] >= t_ref[:, :1], x_ref[...],
                           jnp.float32(0.0))


def _sc_thr_body(x_hbm, o_hbm, v_v, x_v, t_v, *, k, rpw, d, row0):
    wid = lax.axis_index("s") * 2 + lax.axis_index("c")
    base = row0 + wid * rpw
    nv = d // 16

    def row_body(r, _):
        pltpu.sync_copy(x_hbm.at[base + r], x_v)

        def enc(i, _):
            for u in range(8):
                o = i * 128 + u * 16
                s = lax.bitcast_convert_type(x_v[pl.ds(o, 16)], jnp.int32)
                v_v[pl.ds(o, 16)] = jnp.where(
                    s < 0, s ^ np.int32(0x7FFFFFFF), s)
            return 0

        lax.fori_loop(0, nv // 8, enc, 0)

        def bit_body(b, pu):
            t = pu | lax.shift_left(np.int32(1), np.int32(31) - b)
            pv = t ^ np.int32(-(2**31))

            def cnt_chunk(j, accs):
                a0, a1, a2, a3 = accs
                for u in range(0, 16, 4):
                    base2 = j * 256 + u * 16
                    a0 = a0 + plsc.all_reduce_population_count(
                        v_v[pl.ds(base2, 16)] >= pv)
                    a1 = a1 + plsc.all_reduce_population_count(
                        v_v[pl.ds(base2 + 16, 16)] >= pv)
                    a2 = a2 + plsc.all_reduce_population_count(
                        v_v[pl.ds(base2 + 32, 16)] >= pv)
                    a3 = a3 + plsc.all_reduce_population_count(
                        v_v[pl.ds(base2 + 48, 16)] >= pv)
                return a0, a1, a2, a3

            z = jnp.zeros((16,), jnp.int32)
            a0, a1, a2, a3 = lax.fori_loop(0, nv // 16, cnt_chunk,
                                           (z, z, z, z))
            cnt = (a0 + a1) + (a2 + a3)
            return jnp.where(cnt >= k, t, pu)

        pu = lax.fori_loop(0, 32, bit_body, jnp.zeros((16,), jnp.int32))
        pv = pu ^ np.int32(-(2**31))
        thr_s = jnp.where(pv < 0, pv ^ np.int32(0x7FFFFFFF), pv)
        thr_f = lax.bitcast_convert_type(thr_s, jnp.float32)
        for u in range(8):
            t_v[pl.ds(u * 16, 16)] = thr_f
        pltpu.sync_copy(t_v, o_hbm.at[wid * rpw + r])
        return 0

    lax.fori_loop(0, rpw, row_body, 0)


def kernel(x):
    d = x.shape[-1]
    k = max(1, int(d * (1.0 - SPARSITY)))
    flat = x.reshape(-1, d)
    rows = flat.shape[0]
    tc_rows = rows - SC_ROWS
    tc_blocks = tc_rows // ROW_BLOCK
    sc_blocks = SC_ROWS // ROW_BLOCK

    thr = pl.kernel(
        functools.partial(_sc_thr_body, k=k, rpw=SC_ROWS // _NW, d=d,
                          row0=tc_rows),
        out_type=jax.ShapeDtypeStruct((SC_ROWS, 128), jnp.float32),
        mesh=plsc.VectorSubcoreMesh(core_axis_name="c", subcore_axis_name="s"),
        compiler_params=pltpu.CompilerParams(needs_layout_passes=False),
        scratch_types=[
            pltpu.VMEM((d,), jnp.int32),
            pltpu.VMEM((d,), jnp.float32),
            pltpu.VMEM((128,), jnp.float32),
        ],
    )(flat)

    o1 = pl.pallas_call(
        functools.partial(_topk_mask_kernel, k=k),
        grid=(tc_blocks,),
        in_specs=[pl.BlockSpec((ROW_BLOCK, d), lambda i: (i, 0))],
        out_specs=pl.BlockSpec((ROW_BLOCK, d), lambda i: (i, 0)),
        out_shape=jax.ShapeDtypeStruct((rows, d), jnp.float32),
        compiler_params=pltpu.CompilerParams(
            dimension_semantics=("arbitrary",),
        ),
    )(flat)

    out = pl.pallas_call(
        _apply_thr_kernel,
        grid=(sc_blocks,),
        in_specs=[
            pl.BlockSpec((ROW_BLOCK, d), lambda i, n=tc_blocks: (i + n, 0)),
            pl.BlockSpec((ROW_BLOCK, 128), lambda i: (i, 0)),
            pl.BlockSpec((ROW_BLOCK, d), lambda i, n=tc_blocks: (i + n, 0)),
        ],
        out_specs=pl.BlockSpec((ROW_BLOCK, d),
                               lambda i, n=tc_blocks: (i + n, 0)),
        out_shape=jax.ShapeDtypeStruct((rows, d), jnp.float32),
        input_output_aliases={2: 0},
    )(flat, thr, o1)
    return out.reshape(x.shape)
